# baseline (speedup 1.0000x reference)
import functools

import jax
import jax.numpy as jnp
from jax import lax
from jax.experimental import pallas as pl
from jax.experimental.pallas import tpu as pltpu
from jax.experimental.pallas import tpu_sc as plsc

B = 128
T = 8192
K = 64
L = 16
NV = T // L
NBIN = 256
LW = 0.6
PW = 0.4

_info = plsc.get_sparse_core_info()
NC, NS = _info.num_cores, _info.num_subcores
NW = NC * NS
RPW = B // NW


def _sobel_topk_body(l_hbm, p_hbm, idx_hbm, sal_hbm, xl, xp, el, ep, salb,
                     hist, idxb):
    wid = lax.axis_index("s") * NC + lax.axis_index("c")
    iota = lax.iota(jnp.int32, L)
    zf = jnp.zeros((L,), jnp.float32)
    zi = jnp.zeros((L,), jnp.int32)
    ones = jnp.ones((L,), jnp.int32)

    def edge_pass(x_ref, e_ref):
        def body(i, m):
            a = x_ref[pl.ds(i * L + (L - 1), L)]
            b = x_ref[pl.ds(i * L + (L + 1), L)]
            e = jnp.abs(b - a)
            e_ref[pl.ds(i * L, L)] = e
            return jnp.maximum(m, e)

        m = lax.fori_loop(0, NV, body, zf)
        return jnp.maximum(jnp.broadcast_to(jnp.max(m), (L,)), 1e-8)

    def zero_hist():
        def body(j, _):
            hist[pl.ds(j * L, L)] = zi
            return 0

        lax.fori_loop(0, NBIN, body, 0)

    def scan_hist(g_in):
        def body(ji, carry):
            tot, bkt, g_out = carry
            j = (NBIN - 1) - ji
            cnt = jnp.sum(hist[pl.ds(j * L, L)])
            crossed = (tot < K) & (tot + cnt >= K)
            bkt = jnp.where(crossed, j, bkt)
            g_out = jnp.where(crossed, tot, g_out)
            return tot + cnt, bkt, g_out

        _, bkt, g_out = lax.fori_loop(0, NBIN, body, (g_in, 0, g_in))
        return bkt, g_out

    def row_body(r, _):
        row = wid * RPW + r
        pltpu.sync_copy(l_hbm.at[pl.ds(row * T, T)], xl.at[pl.ds(L, T)])
        pltpu.sync_copy(p_hbm.at[pl.ds(row * T, T)], xp.at[pl.ds(L, T)])
        xl[pl.ds(0, L)] = zf
        xl[pl.ds(T + L, L)] = zf
        xp[pl.ds(0, L)] = zf
        xp[pl.ds(T + L, L)] = zf

        lmax = edge_pass(xl, el)
        pmax = edge_pass(xp, ep)
        wl = jnp.full((L,), LW, jnp.float32) / lmax
        wp = jnp.full((L,), PW, jnp.float32) / pmax

        def blend(i, _):
            s = el[pl.ds(i * L, L)] * wl + ep[pl.ds(i * L, L)] * wp
            salb[pl.ds(i * L, L)] = s
            return 0

        lax.fori_loop(0, NV, blend, 0)
        pltpu.sync_copy(salb, sal_hbm.at[pl.ds(row * T, T)])

        def hist_pass(pshift, pref, shift, dmask):
            def body(i, _):
                u = plsc.bitcast(salb[pl.ds(i * L, L)], jnp.int32)
                d = lax.shift_right_logical(u, shift) & dmask
                m = (lax.shift_right_logical(u, pshift) == pref
                     if pshift < 32 else None)
                plsc.addupdate_scatter(hist, [d * L + iota], ones, mask=m)
                return 0

            lax.fori_loop(0, NV, body, 0)

        pref = jnp.int32(0)
        g = jnp.int32(0)
        for pshift, shift, bits in ((32, 23, 9), (23, 15, 8), (15, 7, 8),
                                    (7, 0, 7)):
            zero_hist()
            hist_pass(pshift, pref, shift, (1 << bits) - 1)
            bkt, g = scan_hist(g)
            pref = pref * (1 << bits) + bkt

        v = pref
        need = K - g

        def collect(i, carry):
            o_sel, o_eq = carry
            u = plsc.bitcast(salb[pl.ds(i * L, L)], jnp.int32)
            gt = u > v
            eq = u == v
            eq_i = lax.select(eq, ones, zi)
            eq_rank = o_eq + plsc.cumsum(eq_i) - 1
            sel = gt | (eq & (eq_rank < need))
            sel_i = lax.select(sel, ones, zi)
            pos = o_sel + plsc.cumsum(sel_i) - 1
            plsc.store_scatter(idxb, [pos], i * L + iota, mask=sel)
            return o_sel + jnp.sum(sel_i), o_eq + jnp.sum(eq_i)

        lax.fori_loop(0, NV, collect, (jnp.int32(0), jnp.int32(0)))
        pltpu.sync_copy(idxb, idx_hbm.at[pl.ds(row * K, K)])
        return 0

    lax.fori_loop(0, RPW, row_body, 0)


@functools.partial(
    pl.kernel,
    out_type=(
        jax.ShapeDtypeStruct((B * K,), jnp.int32),
        jax.ShapeDtypeStruct((B * T,), jnp.float32),
    ),
    mesh=plsc.VectorSubcoreMesh(core_axis_name="c", subcore_axis_name="s"),
    compiler_params=pltpu.CompilerParams(needs_layout_passes=False),
    scratch_types=(
        pltpu.VMEM((T + 2 * L,), jnp.float32),
        pltpu.VMEM((T + 2 * L,), jnp.float32),
        pltpu.VMEM((T,), jnp.float32),
        pltpu.VMEM((T,), jnp.float32),
        pltpu.VMEM((T,), jnp.float32),
        pltpu.VMEM((NBIN * L,), jnp.int32),
        pltpu.VMEM((K,), jnp.int32),
    ),
)
def _sobel_topk(*args):
    _sobel_topk_body(*args)


def kernel(loudness, pitch):
    topk_idx, saliency = _sobel_topk(loudness.reshape(-1), pitch.reshape(-1))
    mask = jnp.zeros((B, K), dtype=jnp.bool_)
    return topk_idx.reshape(B, K), saliency.reshape(B, T), mask


# speedup vs baseline: 1.1742x; 1.1742x over previous
import functools

import jax
import jax.numpy as jnp
from jax import lax
from jax.experimental import pallas as pl
from jax.experimental.pallas import tpu as pltpu
from jax.experimental.pallas import tpu_sc as plsc

B = 128
T = 8192
K = 64
L = 16
NV = T // L
NBIN = 256
NBIN1 = 128
LW = 0.6
PW = 0.4

_info = plsc.get_sparse_core_info()
NC, NS = _info.num_cores, _info.num_subcores
NW = NC * NS
RPW = B // NW


def _sobel_topk_body(l_hbm, p_hbm, idx_hbm, sal_hbm, xl, xp, el, ep, salb,
                     hist, idxb, thr, sem):
    wid = lax.axis_index("s") * NC + lax.axis_index("c")
    iota = lax.iota(jnp.int32, L)
    zf = jnp.zeros((L,), jnp.float32)
    zi = jnp.zeros((L,), jnp.int32)
    ones = jnp.ones((L,), jnp.int32)

    def zero_hist(nbin):
        def body(j, _):
            hist[pl.ds(j * L, L)] = zi
            return 0

        lax.fori_loop(0, nbin, body, 0, unroll=8)

    def scan_hist(g_in, nbin):
        def body(ji, carry):
            tot, bkt, g_out, h_out = carry
            j = (nbin - 1) - ji
            cnt = jnp.sum(hist[pl.ds(j * L, L)])
            crossed = (tot < K) & (tot + cnt >= K)
            bkt = jnp.where(crossed, j, bkt)
            g_out = jnp.where(crossed, tot, g_out)
            h_out = jnp.where(crossed, cnt, h_out)
            return tot + cnt, bkt, g_out, h_out

        _, bkt, g_out, h_out = lax.fori_loop(
            0, nbin, body, (g_in, 0, g_in, jnp.int32(0)), unroll=4)
        return bkt, g_out, h_out

    def hist_pass(pshift, pref, shift, dmask):
        def body(i, _):
            u = plsc.bitcast(salb[pl.ds(i * L, L)], jnp.int32)
            d = lax.shift_right_logical(u, shift) & dmask
            m = lax.shift_right_logical(u, pshift) == pref
            plsc.addupdate_scatter(hist, [d * L + iota], ones, mask=m)
            return 0

        lax.fori_loop(0, NV, body, 0, unroll=4)

    def row_body(r, _):
        row = wid * RPW + r
        pltpu.sync_copy(l_hbm.at[pl.ds(row * T, T)], xl.at[pl.ds(L, T)])
        pltpu.sync_copy(p_hbm.at[pl.ds(row * T, T)], xp.at[pl.ds(L, T)])
        xl[pl.ds(0, L)] = zf
        xl[pl.ds(T + L, L)] = zf
        xp[pl.ds(0, L)] = zf
        xp[pl.ds(T + L, L)] = zf

        def edges(i, carry):
            ml, mp = carry
            e1 = jnp.abs(xl[pl.ds(i * L + (L + 1), L)]
                         - xl[pl.ds(i * L + (L - 1), L)])
            el[pl.ds(i * L, L)] = e1
            e2 = jnp.abs(xp[pl.ds(i * L + (L + 1), L)]
                         - xp[pl.ds(i * L + (L - 1), L)])
            ep[pl.ds(i * L, L)] = e2
            return jnp.maximum(ml, e1), jnp.maximum(mp, e2)

        ml, mp = lax.fori_loop(0, NV, edges, (zf, zf), unroll=4)
        lmax = jnp.maximum(jnp.broadcast_to(jnp.max(ml), (L,)), 1e-8)
        pmax = jnp.maximum(jnp.broadcast_to(jnp.max(mp), (L,)), 1e-8)
        wl = jnp.full((L,), LW, jnp.float32) / lmax
        wp = jnp.full((L,), PW, jnp.float32) / pmax

        zero_hist(NBIN1)

        def blend(i, _):
            s = el[pl.ds(i * L, L)] * wl + ep[pl.ds(i * L, L)] * wp
            salb[pl.ds(i * L, L)] = s
            d = lax.shift_right_logical(plsc.bitcast(s, jnp.int32), 23)
            plsc.addupdate_scatter(hist, [d * L + iota], ones)
            return 0

        lax.fori_loop(0, NV, blend, 0, unroll=4)
        sal_cp = pltpu.async_copy(salb, sal_hbm.at[pl.ds(row * T, T)], sem)

        c1, g1, _ = scan_hist(jnp.int32(0), NBIN1)

        zero_hist(NBIN)
        hist_pass(23, c1, 15, 0xFF)
        c2, g2, h2 = scan_hist(g1, NBIN)
        pref2 = c1 * 256 + c2
        need2 = K - g2

        thr[0] = (pref2 << 15) - 1
        thr[1] = 0

        @pl.when(h2 > need2)
        def _refine():
            zero_hist(NBIN)
            hist_pass(15, pref2, 7, 0xFF)
            c3, g3, _ = scan_hist(g2, NBIN)
            pref3 = pref2 * 256 + c3
            zero_hist(NBIN)
            hist_pass(7, pref3, 0, 0x7F)
            c4, g4, _ = scan_hist(g3, NBIN)
            thr[0] = pref3 * 128 + c4
            thr[1] = K - g4

        v = thr[0]
        need = thr[1]

        def collect(i, carry):
            o_sel, o_eq = carry
            u = plsc.bitcast(salb[pl.ds(i * L, L)], jnp.int32)
            gt = u > v
            eq = u == v
            eq_i = lax.select(eq, ones, zi)
            eq_rank = o_eq + plsc.cumsum(eq_i) - 1
            sel = gt | (eq & (eq_rank < need))
            sel_i = lax.select(sel, ones, zi)
            pos = o_sel + plsc.cumsum(sel_i) - 1
            plsc.store_scatter(idxb, [pos], i * L + iota, mask=sel)
            return o_sel + jnp.sum(sel_i), o_eq + jnp.sum(eq_i)

        lax.fori_loop(0, NV, collect, (jnp.int32(0), jnp.int32(0)), unroll=2)
        sal_cp.wait()
        pltpu.sync_copy(idxb, idx_hbm.at[pl.ds(row * K, K)])
        return 0

    lax.fori_loop(0, RPW, row_body, 0)


@functools.partial(
    pl.kernel,
    out_type=(
        jax.ShapeDtypeStruct((B * K,), jnp.int32),
        jax.ShapeDtypeStruct((B * T,), jnp.float32),
    ),
    mesh=plsc.VectorSubcoreMesh(core_axis_name="c", subcore_axis_name="s"),
    compiler_params=pltpu.CompilerParams(needs_layout_passes=False),
    scratch_types=(
        pltpu.VMEM((T + 2 * L,), jnp.float32),
        pltpu.VMEM((T + 2 * L,), jnp.float32),
        pltpu.VMEM((T,), jnp.float32),
        pltpu.VMEM((T,), jnp.float32),
        pltpu.VMEM((T,), jnp.float32),
        pltpu.VMEM((NBIN * L,), jnp.int32),
        pltpu.VMEM((K,), jnp.int32),
        pltpu.SMEM((2,), jnp.int32),
        pltpu.SemaphoreType.DMA,
    ),
)
def _sobel_topk(*args):
    _sobel_topk_body(*args)


def kernel(loudness, pitch):
    topk_idx, saliency = _sobel_topk(loudness.reshape(-1), pitch.reshape(-1))
    mask = jnp.zeros((B, K), dtype=jnp.bool_)
    return topk_idx.reshape(B, K), saliency.reshape(B, T), mask


# speedup vs baseline: 1.6409x; 1.3974x over previous
import functools

import jax
import jax.numpy as jnp
from jax import lax
from jax.experimental import pallas as pl
from jax.experimental.pallas import tpu as pltpu
from jax.experimental.pallas import tpu_sc as plsc

B = 128
T = 8192
K = 64
L = 16
NV = T // L
NBIN = 256
NBIN1 = 128
LW = 0.6
PW = 0.4

_info = plsc.get_sparse_core_info()
NC, NS = _info.num_cores, _info.num_subcores
NW = NC * NS
RPW = B // NW


def _sobel_topk_body(l_hbm, p_hbm, idx_hbm, sal_hbm, xl, xp, el, ep, salb,
                     cidx, cval, hist, idxb, thr, sem):
    wid = lax.axis_index("s") * NC + lax.axis_index("c")
    iota = lax.iota(jnp.int32, L)
    zf = jnp.zeros((L,), jnp.float32)
    zi = jnp.zeros((L,), jnp.int32)
    ones = jnp.ones((L,), jnp.int32)

    def zero_hist(nbin):
        def body(j, _):
            hist[pl.ds(j * L, L)] = zi
            return 0

        lax.fori_loop(0, nbin, body, 0, unroll=8)

    def scan_hist(g_in, nbin):
        ng = nbin // 16

        def g_body(jg, carry):
            tot, grp, g_at = carry
            j = (ng - 1) - jg

            def inner(b, acc):
                return acc + hist[pl.ds((j * 16 + b) * L, L)]

            cnt = jnp.sum(lax.fori_loop(0, 16, inner, zi, unroll=8))
            crossed = (tot < K) & (tot + cnt >= K)
            grp = jnp.where(crossed, j, grp)
            g_at = jnp.where(crossed, tot, g_at)
            return tot + cnt, grp, g_at

        _, grp, g_at = lax.fori_loop(0, ng, g_body,
                                     (g_in, jnp.int32(0), g_in))

        def b_body(bb, carry):
            tot, bkt, g_out, h_out = carry
            b = 15 - bb
            cnt = jnp.sum(hist[pl.ds((grp * 16 + b) * L, L)])
            crossed = (tot < K) & (tot + cnt >= K)
            bkt = jnp.where(crossed, grp * 16 + b, bkt)
            g_out = jnp.where(crossed, tot, g_out)
            h_out = jnp.where(crossed, cnt, h_out)
            return tot + cnt, bkt, g_out, h_out

        _, bkt, g_out, h_out = lax.fori_loop(
            0, 16, b_body, (g_at, jnp.int32(0), g_at, jnp.int32(0)),
            unroll=2)
        return bkt, g_out, h_out

    def cand_hist(ncv, pshift, pref, shift, dmask):
        def body(i, _):
            u = plsc.bitcast(cval[pl.ds(i * L, L)], jnp.int32)
            d = lax.shift_right_logical(u, shift) & dmask
            m = lax.shift_right_logical(u, pshift) == pref
            plsc.addupdate_scatter(hist, [d * L + iota], ones, mask=m)
            return 0

        lax.fori_loop(0, ncv, body, 0)

    def row_body(r, _):
        row = wid * RPW + r
        pltpu.sync_copy(l_hbm.at[pl.ds(row * T, T)], xl.at[pl.ds(L, T)])
        pltpu.sync_copy(p_hbm.at[pl.ds(row * T, T)], xp.at[pl.ds(L, T)])
        xl[pl.ds(0, L)] = zf
        xl[pl.ds(T + L, L)] = zf
        xp[pl.ds(0, L)] = zf
        xp[pl.ds(T + L, L)] = zf

        def edges(i, carry):
            ml, mp = carry
            e1 = jnp.abs(xl[pl.ds(i * L + (L + 1), L)]
                         - xl[pl.ds(i * L + (L - 1), L)])
            el[pl.ds(i * L, L)] = e1
            e2 = jnp.abs(xp[pl.ds(i * L + (L + 1), L)]
                         - xp[pl.ds(i * L + (L - 1), L)])
            ep[pl.ds(i * L, L)] = e2
            return jnp.maximum(ml, e1), jnp.maximum(mp, e2)

        ml, mp = lax.fori_loop(0, NV, edges, (zf, zf), unroll=4)
        lmax = jnp.maximum(jnp.broadcast_to(jnp.max(ml), (L,)), 1e-8)
        pmax = jnp.maximum(jnp.broadcast_to(jnp.max(mp), (L,)), 1e-8)
        wl = jnp.full((L,), LW, jnp.float32) / lmax
        wp = jnp.full((L,), PW, jnp.float32) / pmax

        zero_hist(NBIN1)

        def blend(i, _):
            s = el[pl.ds(i * L, L)] * wl + ep[pl.ds(i * L, L)] * wp
            salb[pl.ds(i * L, L)] = s
            d = lax.shift_right_logical(plsc.bitcast(s, jnp.int32), 23)
            plsc.addupdate_scatter(hist, [d * L + iota], ones)
            return 0

        lax.fori_loop(0, NV, blend, 0, unroll=4)
        sal_cp = pltpu.async_copy(salb, sal_hbm.at[pl.ds(row * T, T)], sem)

        c1, g1, _ = scan_hist(jnp.int32(0), NBIN1)

        def compact(i, o_vec):
            u = plsc.bitcast(salb[pl.ds(i * L, L)], jnp.int32)
            m = lax.shift_right_logical(u, 23) >= c1
            pos = o_vec + plsc.cumsum(lax.select(m, ones, zi)) - 1
            plsc.store_scatter(cidx, [pos], i * L + iota, mask=m)
            plsc.store_scatter(cval, [pos], plsc.bitcast(u, jnp.float32),
                               mask=m)
            return o_vec + plsc.all_reduce_population_count(m)

        o_vec = lax.fori_loop(0, NV, compact, zi, unroll=4)
        n = jnp.max(o_vec)
        plsc.store_scatter(cval, [n + iota], zf)
        ncv = lax.shift_right_logical(n + (L - 1), 4)

        zero_hist(NBIN)
        cand_hist(ncv, 23, c1, 15, 0xFF)
        c2, g2, h2 = scan_hist(g1, NBIN)
        pref2 = c1 * 256 + c2
        need2 = K - g2

        thr[0] = (pref2 << 15) - 1
        thr[1] = 0

        @pl.when(h2 > need2)
        def _refine():
            zero_hist(NBIN)
            cand_hist(ncv, 15, pref2, 7, 0xFF)
            c3, g3, _ = scan_hist(g2, NBIN)
            pref3 = pref2 * 256 + c3
            zero_hist(NBIN)
            cand_hist(ncv, 7, pref3, 0, 0x7F)
            c4, g4, _ = scan_hist(g3, NBIN)
            thr[0] = pref3 * 128 + c4
            thr[1] = K - g4

        v = thr[0]
        need = thr[1]

        def collect(i, carry):
            o_sel, o_eq = carry
            u = plsc.bitcast(cval[pl.ds(i * L, L)], jnp.int32)
            gt = u > v
            eq = u == v
            eq_rank = o_eq + plsc.cumsum(lax.select(eq, ones, zi)) - 1
            sel = gt | (eq & (eq_rank < need))
            pos = o_sel + plsc.cumsum(lax.select(sel, ones, zi)) - 1
            plsc.store_scatter(idxb, [pos], cidx[pl.ds(i * L, L)], mask=sel)
            return (o_sel + plsc.all_reduce_population_count(sel),
                    o_eq + plsc.all_reduce_population_count(eq))

        lax.fori_loop(0, ncv, collect, (zi, zi))
        sal_cp.wait()
        pltpu.sync_copy(idxb, idx_hbm.at[pl.ds(row * K, K)])
        return 0

    lax.fori_loop(0, RPW, row_body, 0)


@functools.partial(
    pl.kernel,
    out_type=(
        jax.ShapeDtypeStruct((B * K,), jnp.int32),
        jax.ShapeDtypeStruct((B * T,), jnp.float32),
    ),
    mesh=plsc.VectorSubcoreMesh(core_axis_name="c", subcore_axis_name="s"),
    compiler_params=pltpu.CompilerParams(needs_layout_passes=False),
    scratch_types=(
        pltpu.VMEM((T + 2 * L,), jnp.float32),
        pltpu.VMEM((T + 2 * L,), jnp.float32),
        pltpu.VMEM((T,), jnp.float32),
        pltpu.VMEM((T,), jnp.float32),
        pltpu.VMEM((T,), jnp.float32),
        pltpu.VMEM((T + L,), jnp.int32),
        pltpu.VMEM((T + L,), jnp.float32),
        pltpu.VMEM((NBIN * L,), jnp.int32),
        pltpu.VMEM((K,), jnp.int32),
        pltpu.SMEM((2,), jnp.int32),
        pltpu.SemaphoreType.DMA,
    ),
)
def _sobel_topk(*args):
    _sobel_topk_body(*args)


def kernel(loudness, pitch):
    topk_idx, saliency = _sobel_topk(loudness.reshape(-1), pitch.reshape(-1))
    mask = jnp.zeros((B, K), dtype=jnp.bool_)
    return topk_idx.reshape(B, K), saliency.reshape(B, T), mask


# speedup vs baseline: 2.4492x; 1.4926x over previous
import functools

import jax
import jax.numpy as jnp
from jax import lax
from jax.experimental import pallas as pl
from jax.experimental.pallas import tpu as pltpu
from jax.experimental.pallas import tpu_sc as plsc

B = 128
T = 8192
K = 64
L = 16
NV = T // L
NBIN = 256
NBIN1 = 128
LW = 0.6
PW = 0.4

_info = plsc.get_sparse_core_info()
NC, NS = _info.num_cores, _info.num_subcores
NW = NC * NS
RPW = B // NW


def _sobel_topk_body(l_hbm, p_hbm, idx_hbm, sal_hbm, xl0, xp0, xl1, xp1,
                     el, ep, salb, cidx, cval, hist, idxb, thr, sem,
                     sem_in0, sem_in1):
    wid = lax.axis_index("s") * NC + lax.axis_index("c")
    iota = lax.iota(jnp.int32, L)
    zf = jnp.zeros((L,), jnp.float32)
    zi = jnp.zeros((L,), jnp.int32)
    ones = jnp.ones((L,), jnp.int32)

    def zero_hist(nbin):
        def body(j, _):
            hist[pl.ds(j * L, L)] = zi
            return 0

        lax.fori_loop(0, nbin, body, 0, unroll=8)

    def scan_hist(g_in, nbin):
        ng = nbin // 16

        def g_body(jg, carry):
            tot, grp, g_at = carry
            j = (ng - 1) - jg

            def inner(b, acc):
                return acc + hist[pl.ds((j * 16 + b) * L, L)]

            cnt = jnp.sum(lax.fori_loop(0, 16, inner, zi, unroll=8))
            crossed = (tot < K) & (tot + cnt >= K)
            grp = jnp.where(crossed, j, grp)
            g_at = jnp.where(crossed, tot, g_at)
            return tot + cnt, grp, g_at

        _, grp, g_at = lax.fori_loop(0, ng, g_body,
                                     (g_in, jnp.int32(0), g_in))

        def b_body(bb, carry):
            tot, bkt, g_out, h_out = carry
            b = 15 - bb
            cnt = jnp.sum(hist[pl.ds((grp * 16 + b) * L, L)])
            crossed = (tot < K) & (tot + cnt >= K)
            bkt = jnp.where(crossed, grp * 16 + b, bkt)
            g_out = jnp.where(crossed, tot, g_out)
            h_out = jnp.where(crossed, cnt, h_out)
            return tot + cnt, bkt, g_out, h_out

        _, bkt, g_out, h_out = lax.fori_loop(
            0, 16, b_body, (g_at, jnp.int32(0), g_at, jnp.int32(0)),
            unroll=2)
        return bkt, g_out, h_out

    def cand_hist(ncv, pshift, pref, shift, dmask):
        def body(i, _):
            u = plsc.bitcast(cval[pl.ds(i * L, L)], jnp.int32)
            d = lax.shift_right_logical(u, shift) & dmask
            m = lax.shift_right_logical(u, pshift) == pref
            plsc.addupdate_scatter(hist, [d * L + iota], ones, mask=m)
            return 0

        lax.fori_loop(0, ncv, body, 0)

    def start_row_in(r, xl, xp, sem_in):
        row = wid * RPW + r
        cl = pltpu.async_copy(l_hbm.at[pl.ds(row * T, T)],
                              xl.at[pl.ds(L, T)], sem_in)
        cp = pltpu.async_copy(p_hbm.at[pl.ds(row * T, T)],
                              xp.at[pl.ds(L, T)], sem_in)
        return cl, cp

    def row_body(r, xl, xp):
        row = wid * RPW + r
        xl[pl.ds(0, L)] = zf
        xl[pl.ds(T + L, L)] = zf
        xp[pl.ds(0, L)] = zf
        xp[pl.ds(T + L, L)] = zf

        @plsc.parallel_loop(0, NV, unroll=4, carry=(zf, zf))
        def edge_loop(i, carry):
            ml, mp = carry
            e1 = jnp.abs(xl[pl.ds(i * L + (L + 1), L)]
                         - xl[pl.ds(i * L + (L - 1), L)])
            el[pl.ds(i * L, L)] = e1
            e2 = jnp.abs(xp[pl.ds(i * L + (L + 1), L)]
                         - xp[pl.ds(i * L + (L - 1), L)])
            ep[pl.ds(i * L, L)] = e2
            return jnp.maximum(ml, e1), jnp.maximum(mp, e2)

        ml, mp = edge_loop
        lmax = jnp.maximum(jnp.broadcast_to(jnp.max(ml), (L,)), 1e-8)
        pmax = jnp.maximum(jnp.broadcast_to(jnp.max(mp), (L,)), 1e-8)
        wl = jnp.full((L,), LW, jnp.float32) / lmax
        wp = jnp.full((L,), PW, jnp.float32) / pmax

        zero_hist(NBIN1)

        def blend(i, _):
            s = el[pl.ds(i * L, L)] * wl + ep[pl.ds(i * L, L)] * wp
            salb[pl.ds(i * L, L)] = s
            d = lax.shift_right_logical(plsc.bitcast(s, jnp.int32), 23)
            plsc.addupdate_scatter(hist, [d * L + iota], ones)
            return 0

        lax.fori_loop(0, NV, blend, 0, unroll=4)
        sal_cp = pltpu.async_copy(salb, sal_hbm.at[pl.ds(row * T, T)], sem)

        c1, g1, _ = scan_hist(jnp.int32(0), NBIN1)

        @plsc.parallel_loop(0, NV, unroll=4, carry=zi)
        def compact_loop(i, o_vec):
            u = plsc.bitcast(salb[pl.ds(i * L, L)], jnp.int32)
            m = lax.shift_right_logical(u, 23) >= c1
            pos = o_vec + plsc.cumsum(lax.select(m, ones, zi)) - 1
            plsc.store_scatter(cidx, [pos], i * L + iota, mask=m)
            plsc.store_scatter(cval, [pos], plsc.bitcast(u, jnp.float32),
                               mask=m)
            return o_vec + plsc.all_reduce_population_count(m)

        o_vec = compact_loop
        n = jnp.max(o_vec)
        plsc.store_scatter(cval, [n + iota], zf)
        ncv = lax.shift_right_logical(n + (L - 1), 4)

        zero_hist(NBIN)
        cand_hist(ncv, 23, c1, 15, 0xFF)
        c2, g2, h2 = scan_hist(g1, NBIN)
        pref2 = c1 * 256 + c2
        need2 = K - g2

        thr[0] = (pref2 << 15) - 1
        thr[1] = 0

        @pl.when(h2 > need2)
        def _refine():
            zero_hist(NBIN)
            cand_hist(ncv, 15, pref2, 7, 0xFF)
            c3, g3, _ = scan_hist(g2, NBIN)
            pref3 = pref2 * 256 + c3
            zero_hist(NBIN)
            cand_hist(ncv, 7, pref3, 0, 0x7F)
            c4, g4, _ = scan_hist(g3, NBIN)
            thr[0] = pref3 * 128 + c4
            thr[1] = K - g4

        v = thr[0]
        need = thr[1]

        def collect(i, carry):
            o_sel, o_eq = carry
            u = plsc.bitcast(cval[pl.ds(i * L, L)], jnp.int32)
            gt = u > v
            eq = u == v
            eq_rank = o_eq + plsc.cumsum(lax.select(eq, ones, zi)) - 1
            sel = gt | (eq & (eq_rank < need))
            pos = o_sel + plsc.cumsum(lax.select(sel, ones, zi)) - 1
            plsc.store_scatter(idxb, [pos], cidx[pl.ds(i * L, L)], mask=sel)
            return (o_sel + plsc.all_reduce_population_count(sel),
                    o_eq + plsc.all_reduce_population_count(eq))

        lax.fori_loop(0, ncv, collect, (zi, zi))
        sal_cp.wait()
        pltpu.sync_copy(idxb, idx_hbm.at[pl.ds(row * K, K)])

    bufs = ((xl0, xp0, sem_in0), (xl1, xp1, sem_in1))
    pend = start_row_in(0, *bufs[0])
    for r in range(RPW):
        for c in pend:
            c.wait()
        if r + 1 < RPW:
            nxt = start_row_in(r + 1, *bufs[(r + 1) % 2])
        row_body(r, bufs[r % 2][0], bufs[r % 2][1])
        if r + 1 < RPW:
            pend = nxt


@functools.partial(
    pl.kernel,
    out_type=(
        jax.ShapeDtypeStruct((B * K,), jnp.int32),
        jax.ShapeDtypeStruct((B * T,), jnp.float32),
    ),
    mesh=plsc.VectorSubcoreMesh(core_axis_name="c", subcore_axis_name="s"),
    compiler_params=pltpu.CompilerParams(needs_layout_passes=False),
    scratch_types=(
        pltpu.VMEM((T + 2 * L,), jnp.float32),
        pltpu.VMEM((T + 2 * L,), jnp.float32),
        pltpu.VMEM((T + 2 * L,), jnp.float32),
        pltpu.VMEM((T + 2 * L,), jnp.float32),
        pltpu.VMEM((T,), jnp.float32),
        pltpu.VMEM((T,), jnp.float32),
        pltpu.VMEM((T,), jnp.float32),
        pltpu.VMEM((T + L,), jnp.int32),
        pltpu.VMEM((T + L,), jnp.float32),
        pltpu.VMEM((NBIN * L,), jnp.int32),
        pltpu.VMEM((K,), jnp.int32),
        pltpu.SMEM((2,), jnp.int32),
        pltpu.SemaphoreType.DMA,
        pltpu.SemaphoreType.DMA,
        pltpu.SemaphoreType.DMA,
    ),
)
def _sobel_topk(*args):
    _sobel_topk_body(*args)


def kernel(loudness, pitch):
    topk_idx, saliency = _sobel_topk(loudness.reshape(-1), pitch.reshape(-1))
    mask = jnp.zeros((B, K), dtype=jnp.bool_)
    return topk_idx.reshape(B, K), saliency.reshape(B, T), mask


# speedup vs baseline: 3.0866x; 1.2602x over previous
import functools

import jax
import jax.numpy as jnp
from jax import lax
from jax.experimental import pallas as pl
from jax.experimental.pallas import tpu as pltpu
from jax.experimental.pallas import tpu_sc as plsc

B = 128
T = 8192
K = 64
L = 16
NV = T // L
NBIN = 256
NBIN1 = 128
LW = 0.6
PW = 0.4

_info = plsc.get_sparse_core_info()
NC, NS = _info.num_cores, _info.num_subcores
NW = NC * NS
RPW = B // NW


def _sobel_topk_body(l_hbm, p_hbm, idx_hbm, sal_hbm, xl0, xp0, xl1, xp1,
                     el, ep, salb, cidx, cval, hist, idxb, thr, sem,
                     sem_in0, sem_in1, sem_out0, sem_out1):
    sem_out = (sem_out0, sem_out1)
    wid = lax.axis_index("s") * NC + lax.axis_index("c")
    iota = lax.iota(jnp.int32, L)
    zf = jnp.zeros((L,), jnp.float32)
    zi = jnp.zeros((L,), jnp.int32)
    ones = jnp.ones((L,), jnp.int32)

    def zero_hist(nbin):
        @plsc.parallel_loop(0, nbin, unroll=8)
        def body(j):
            hist[pl.ds(j * L, L)] = zi

    def scan_hist(g_in, nbin):
        ng = nbin // 16

        def g_body(jg, carry):
            tot, grp, g_at = carry
            j = (ng - 1) - jg
            vs = [hist[pl.ds((j * 16 + b) * L, L)] for b in range(16)]
            while len(vs) > 1:
                vs = [a + b for a, b in zip(vs[::2], vs[1::2])]
            cnt = jnp.sum(vs[0])
            crossed = (tot < K) & (tot + cnt >= K)
            grp = jnp.where(crossed, j, grp)
            g_at = jnp.where(crossed, tot, g_at)
            return tot + cnt, grp, g_at

        _, grp, g_at = lax.fori_loop(0, ng, g_body,
                                     (g_in, jnp.int32(0), g_in))

        def b_body(bb, carry):
            tot, bkt, g_out, h_out = carry
            b = 15 - bb
            cnt = jnp.sum(hist[pl.ds((grp * 16 + b) * L, L)])
            crossed = (tot < K) & (tot + cnt >= K)
            bkt = jnp.where(crossed, grp * 16 + b, bkt)
            g_out = jnp.where(crossed, tot, g_out)
            h_out = jnp.where(crossed, cnt, h_out)
            return tot + cnt, bkt, g_out, h_out

        _, bkt, g_out, h_out = lax.fori_loop(
            0, 16, b_body, (g_at, jnp.int32(0), g_at, jnp.int32(0)),
            unroll=2)
        return bkt, g_out, h_out

    def cand_hist(ncv, pshift, pref, shift, dmask):
        def body(i, _):
            u = plsc.bitcast(cval[pl.ds(i * L, L)], jnp.int32)
            d = lax.shift_right_logical(u, shift) & dmask
            m = lax.shift_right_logical(u, pshift) == pref
            plsc.addupdate_scatter(hist, [d * L + iota], ones, mask=m)
            return 0

        lax.fori_loop(0, ncv, body, 0)

    def start_row_in(r, xl, xp, sem_in):
        row = wid * RPW + r
        cl = pltpu.async_copy(l_hbm.at[pl.ds(row * T, T)],
                              xl.at[pl.ds(L, T)], sem_in)
        cp = pltpu.async_copy(p_hbm.at[pl.ds(row * T, T)],
                              xp.at[pl.ds(L, T)], sem_in)
        return cl, cp

    def row_body(r, xl, xp):
        row = wid * RPW + r
        xl[pl.ds(0, L)] = zf
        xl[pl.ds(T + L, L)] = zf
        xp[pl.ds(0, L)] = zf
        xp[pl.ds(T + L, L)] = zf

        @plsc.parallel_loop(0, NV, unroll=4, carry=(zf, zf))
        def edge_loop(i, carry):
            ml, mp = carry
            e1 = jnp.abs(xl[pl.ds(i * L + (L + 1), L)]
                         - xl[pl.ds(i * L + (L - 1), L)])
            el[pl.ds(i * L, L)] = e1
            e2 = jnp.abs(xp[pl.ds(i * L + (L + 1), L)]
                         - xp[pl.ds(i * L + (L - 1), L)])
            ep[pl.ds(i * L, L)] = e2
            return jnp.maximum(ml, e1), jnp.maximum(mp, e2)

        ml, mp = edge_loop
        lmax = jnp.maximum(jnp.broadcast_to(jnp.max(ml), (L,)), 1e-8)
        pmax = jnp.maximum(jnp.broadcast_to(jnp.max(mp), (L,)), 1e-8)
        wl = jnp.full((L,), LW, jnp.float32) / lmax
        wp = jnp.full((L,), PW, jnp.float32) / pmax

        zero_hist(NBIN1)

        @plsc.parallel_loop(0, NV, unroll=4)
        def blend(i):
            s = el[pl.ds(i * L, L)] * wl + ep[pl.ds(i * L, L)] * wp
            salb[pl.ds(i * L, L)] = s
            d = lax.shift_right_logical(plsc.bitcast(s, jnp.int32), 23)
            plsc.addupdate_scatter(hist, [d * L + iota], ones)
        sal_cp = pltpu.async_copy(salb, sal_hbm.at[pl.ds(row * T, T)], sem)

        c1, g1, _ = scan_hist(jnp.int32(0), NBIN1)

        @plsc.parallel_loop(0, NV, unroll=4, carry=zi)
        def compact_loop(i, o_vec):
            u = plsc.bitcast(salb[pl.ds(i * L, L)], jnp.int32)
            m = lax.shift_right_logical(u, 23) >= c1
            pos = o_vec + plsc.cumsum(lax.select(m, ones, zi)) - 1
            plsc.store_scatter(cidx, [pos], i * L + iota, mask=m)
            plsc.store_scatter(cval, [pos], plsc.bitcast(u, jnp.float32),
                               mask=m)
            return o_vec + plsc.all_reduce_population_count(m)

        o_vec = compact_loop
        n = jnp.max(o_vec)
        plsc.store_scatter(cval, [n + iota], zf)
        ncv = lax.shift_right_logical(n + (L - 1), 4)

        zero_hist(NBIN)
        cand_hist(ncv, 23, c1, 15, 0xFF)
        c2, g2, h2 = scan_hist(g1, NBIN)
        pref2 = c1 * 256 + c2
        need2 = K - g2

        thr[0] = (pref2 << 15) - 1
        thr[1] = 0

        @pl.when(h2 > need2)
        def _refine():
            zero_hist(NBIN)
            cand_hist(ncv, 15, pref2, 7, 0xFF)
            c3, g3, _ = scan_hist(g2, NBIN)
            pref3 = pref2 * 256 + c3
            zero_hist(NBIN)
            cand_hist(ncv, 7, pref3, 0, 0x7F)
            c4, g4, _ = scan_hist(g3, NBIN)
            thr[0] = pref3 * 128 + c4
            thr[1] = K - g4

        v = thr[0]
        need = thr[1]

        def collect(i, carry):
            o_sel, o_eq = carry
            u = plsc.bitcast(cval[pl.ds(i * L, L)], jnp.int32)
            gt = u > v
            eq = u == v
            eq_rank = o_eq + plsc.cumsum(lax.select(eq, ones, zi)) - 1
            sel = gt | (eq & (eq_rank < need))
            pos = o_sel + plsc.cumsum(lax.select(sel, ones, zi)) - 1
            plsc.store_scatter(idxb.at[r % 2], [pos], cidx[pl.ds(i * L, L)],
                               mask=sel)
            return (o_sel + plsc.all_reduce_population_count(sel),
                    o_eq + plsc.all_reduce_population_count(eq))

        lax.fori_loop(0, ncv, collect, (zi, zi))
        sal_cp.wait()
        return pltpu.async_copy(idxb.at[r % 2],
                                idx_hbm.at[pl.ds(row * K, K)],
                                sem_out[r % 2])

    bufs = ((xl0, xp0, sem_in0), (xl1, xp1, sem_in1))
    pend = start_row_in(0, *bufs[0])
    out_pend = [None, None]
    for r in range(RPW):
        for c in pend:
            c.wait()
        if r + 1 < RPW:
            nxt = start_row_in(r + 1, *bufs[(r + 1) % 2])
        if out_pend[r % 2] is not None:
            out_pend[r % 2].wait()
        out_pend[r % 2] = row_body(r, bufs[r % 2][0], bufs[r % 2][1])
        if r + 1 < RPW:
            pend = nxt
    for d in out_pend:
        d.wait()


@functools.partial(
    pl.kernel,
    out_type=(
        jax.ShapeDtypeStruct((B * K,), jnp.int32),
        jax.ShapeDtypeStruct((B * T,), jnp.float32),
    ),
    mesh=plsc.VectorSubcoreMesh(core_axis_name="c", subcore_axis_name="s"),
    compiler_params=pltpu.CompilerParams(needs_layout_passes=False),
    scratch_types=(
        pltpu.VMEM((T + 2 * L,), jnp.float32),
        pltpu.VMEM((T + 2 * L,), jnp.float32),
        pltpu.VMEM((T + 2 * L,), jnp.float32),
        pltpu.VMEM((T + 2 * L,), jnp.float32),
        pltpu.VMEM((T,), jnp.float32),
        pltpu.VMEM((T,), jnp.float32),
        pltpu.VMEM((T,), jnp.float32),
        pltpu.VMEM((T + L,), jnp.int32),
        pltpu.VMEM((T + L,), jnp.float32),
        pltpu.VMEM((NBIN * L,), jnp.int32),
        pltpu.VMEM((2, K), jnp.int32),
        pltpu.SMEM((2,), jnp.int32),
        pltpu.SemaphoreType.DMA,
        pltpu.SemaphoreType.DMA,
        pltpu.SemaphoreType.DMA,
        pltpu.SemaphoreType.DMA,
        pltpu.SemaphoreType.DMA,
    ),
)
def _sobel_topk(*args):
    _sobel_topk_body(*args)


def kernel(loudness, pitch):
    topk_idx, saliency = _sobel_topk(loudness.reshape(-1), pitch.reshape(-1))
    mask = jnp.zeros((B, K), dtype=jnp.bool_)
    return topk_idx.reshape(B, K), saliency.reshape(B, T), mask


# speedup vs baseline: 3.2822x; 1.0634x over previous
import functools

import jax
import jax.numpy as jnp
from jax import lax
from jax.experimental import pallas as pl
from jax.experimental.pallas import tpu as pltpu
from jax.experimental.pallas import tpu_sc as plsc

B = 128
T = 8192
K = 64
L = 16
NV = T // L
NBIN = 256
NBIN1 = 128
LW = 0.6
PW = 0.4

_info = plsc.get_sparse_core_info()
NC, NS = _info.num_cores, _info.num_subcores
NW = NC * NS
RPW = B // NW


def _sobel_topk_body(l_hbm, p_hbm, idx_hbm, sal_hbm, xl0, xp0, xl1, xp1,
                     el, ep, salb, cidx, cval, hist, idxb, thr, sem,
                     sem_in0, sem_in1, sem_out0, sem_out1):
    sem_out = (sem_out0, sem_out1)
    wid = lax.axis_index("s") * NC + lax.axis_index("c")
    iota = lax.iota(jnp.int32, L)
    zf = jnp.zeros((L,), jnp.float32)
    zi = jnp.zeros((L,), jnp.int32)
    ones = jnp.ones((L,), jnp.int32)

    def zero_hist(nbin):
        @plsc.parallel_loop(0, nbin, unroll=8)
        def body(j):
            hist[pl.ds(j * L, L)] = zi

    def scan_hist(g_in, nbin):
        ng = nbin // 16

        def g_body(jg, carry):
            tot, grp, g_at = carry
            j = (ng - 1) - jg
            vs = [hist[pl.ds((j * 16 + b) * L, L)] for b in range(16)]
            while len(vs) > 1:
                vs = [a + b for a, b in zip(vs[::2], vs[1::2])]
            cnt = jnp.sum(vs[0])
            crossed = (tot < K) & (tot + cnt >= K)
            grp = jnp.where(crossed, j, grp)
            g_at = jnp.where(crossed, tot, g_at)
            return tot + cnt, grp, g_at

        _, grp, g_at = lax.fori_loop(0, ng, g_body,
                                     (g_in, jnp.int32(0), g_in))

        def b_body(bb, carry):
            tot, bkt, g_out, h_out = carry
            b = 15 - bb
            cnt = jnp.sum(hist[pl.ds((grp * 16 + b) * L, L)])
            crossed = (tot < K) & (tot + cnt >= K)
            bkt = jnp.where(crossed, grp * 16 + b, bkt)
            g_out = jnp.where(crossed, tot, g_out)
            h_out = jnp.where(crossed, cnt, h_out)
            return tot + cnt, bkt, g_out, h_out

        _, bkt, g_out, h_out = lax.fori_loop(
            0, 16, b_body, (g_at, jnp.int32(0), g_at, jnp.int32(0)),
            unroll=2)
        return bkt, g_out, h_out

    def cand_hist(ncv, pshift, pref, shift, dmask):
        @plsc.parallel_loop(0, ncv, unroll=2)
        def body(i):
            u = plsc.bitcast(cval[pl.ds(i * L, L)], jnp.int32)
            d = lax.shift_right_logical(u, shift) & dmask
            m = lax.shift_right_logical(u, pshift) == pref
            plsc.addupdate_scatter(hist, [d * L + iota], ones, mask=m)

    def start_row_in(r, xl, xp, sem_in):
        row = wid * RPW + r
        cl = pltpu.async_copy(l_hbm.at[pl.ds(row * T, T)],
                              xl.at[pl.ds(L, T)], sem_in)
        cp = pltpu.async_copy(p_hbm.at[pl.ds(row * T, T)],
                              xp.at[pl.ds(L, T)], sem_in)
        return cl, cp

    def row_body(r, xl, xp):
        row = wid * RPW + r
        xl[pl.ds(0, L)] = zf
        xl[pl.ds(T + L, L)] = zf
        xp[pl.ds(0, L)] = zf
        xp[pl.ds(T + L, L)] = zf

        @plsc.parallel_loop(0, NV, unroll=8, carry=(zf, zf))
        def edge_loop(i, carry):
            ml, mp = carry
            e1 = jnp.abs(xl[pl.ds(i * L + (L + 1), L)]
                         - xl[pl.ds(i * L + (L - 1), L)])
            el[pl.ds(i * L, L)] = e1
            e2 = jnp.abs(xp[pl.ds(i * L + (L + 1), L)]
                         - xp[pl.ds(i * L + (L - 1), L)])
            ep[pl.ds(i * L, L)] = e2
            return jnp.maximum(ml, e1), jnp.maximum(mp, e2)

        ml, mp = edge_loop
        lmax = jnp.maximum(jnp.broadcast_to(jnp.max(ml), (L,)), 1e-8)
        pmax = jnp.maximum(jnp.broadcast_to(jnp.max(mp), (L,)), 1e-8)
        wl = jnp.full((L,), LW, jnp.float32) / lmax
        wp = jnp.full((L,), PW, jnp.float32) / pmax

        zero_hist(NBIN1)

        @plsc.parallel_loop(0, NV, unroll=4)
        def blend(i):
            s = el[pl.ds(i * L, L)] * wl + ep[pl.ds(i * L, L)] * wp
            salb[pl.ds(i * L, L)] = s
            d = lax.shift_right_logical(plsc.bitcast(s, jnp.int32), 23)
            plsc.addupdate_scatter(hist, [d * L + iota], ones)
        sal_cp = pltpu.async_copy(salb, sal_hbm.at[pl.ds(row * T, T)], sem)

        c1, g1, _ = scan_hist(jnp.int32(0), NBIN1)

        @plsc.parallel_loop(0, NV, unroll=8, carry=zi)
        def compact_loop(i, o_vec):
            u = plsc.bitcast(salb[pl.ds(i * L, L)], jnp.int32)
            m = lax.shift_right_logical(u, 23) >= c1
            pos = o_vec + plsc.cumsum(lax.select(m, ones, zi)) - 1
            plsc.store_scatter(cidx, [pos], i * L + iota, mask=m)
            plsc.store_scatter(cval, [pos], plsc.bitcast(u, jnp.float32),
                               mask=m)
            return o_vec + plsc.all_reduce_population_count(m)

        o_vec = compact_loop
        n = jnp.max(o_vec)
        plsc.store_scatter(cval, [n + iota], zf)
        ncv = lax.shift_right_logical(n + (L - 1), 4)

        zero_hist(NBIN)
        cand_hist(ncv, 23, c1, 15, 0xFF)
        c2, g2, h2 = scan_hist(g1, NBIN)
        pref2 = c1 * 256 + c2
        need2 = K - g2

        thr[0] = (pref2 << 15) - 1
        thr[1] = 0

        @pl.when(h2 > need2)
        def _refine():
            zero_hist(NBIN)
            cand_hist(ncv, 15, pref2, 7, 0xFF)
            c3, g3, _ = scan_hist(g2, NBIN)
            pref3 = pref2 * 256 + c3
            zero_hist(NBIN)
            cand_hist(ncv, 7, pref3, 0, 0x7F)
            c4, g4, _ = scan_hist(g3, NBIN)
            thr[0] = pref3 * 128 + c4
            thr[1] = K - g4

        v = thr[0]
        need = thr[1]

        def collect(i, carry):
            o_sel, o_eq = carry
            u = plsc.bitcast(cval[pl.ds(i * L, L)], jnp.int32)
            gt = u > v
            eq = u == v
            eq_rank = o_eq + plsc.cumsum(lax.select(eq, ones, zi)) - 1
            sel = gt | (eq & (eq_rank < need))
            pos = o_sel + plsc.cumsum(lax.select(sel, ones, zi)) - 1
            plsc.store_scatter(idxb.at[r % 2], [pos], cidx[pl.ds(i * L, L)],
                               mask=sel)
            return (o_sel + plsc.all_reduce_population_count(sel),
                    o_eq + plsc.all_reduce_population_count(eq))

        lax.fori_loop(0, ncv, collect, (zi, zi))
        sal_cp.wait()
        return pltpu.async_copy(idxb.at[r % 2],
                                idx_hbm.at[pl.ds(row * K, K)],
                                sem_out[r % 2])

    bufs = ((xl0, xp0, sem_in0), (xl1, xp1, sem_in1))
    pend = start_row_in(0, *bufs[0])
    out_pend = [None, None]
    for r in range(RPW):
        for c in pend:
            c.wait()
        if r + 1 < RPW:
            nxt = start_row_in(r + 1, *bufs[(r + 1) % 2])
        if out_pend[r % 2] is not None:
            out_pend[r % 2].wait()
        out_pend[r % 2] = row_body(r, bufs[r % 2][0], bufs[r % 2][1])
        if r + 1 < RPW:
            pend = nxt
    for d in out_pend:
        d.wait()


@functools.partial(
    pl.kernel,
    out_type=(
        jax.ShapeDtypeStruct((B * K,), jnp.int32),
        jax.ShapeDtypeStruct((B * T,), jnp.float32),
    ),
    mesh=plsc.VectorSubcoreMesh(core_axis_name="c", subcore_axis_name="s"),
    compiler_params=pltpu.CompilerParams(needs_layout_passes=False),
    scratch_types=(
        pltpu.VMEM((T + 2 * L,), jnp.float32),
        pltpu.VMEM((T + 2 * L,), jnp.float32),
        pltpu.VMEM((T + 2 * L,), jnp.float32),
        pltpu.VMEM((T + 2 * L,), jnp.float32),
        pltpu.VMEM((T,), jnp.float32),
        pltpu.VMEM((T,), jnp.float32),
        pltpu.VMEM((T,), jnp.float32),
        pltpu.VMEM((T + L,), jnp.int32),
        pltpu.VMEM((T + L,), jnp.float32),
        pltpu.VMEM((NBIN * L,), jnp.int32),
        pltpu.VMEM((2, K), jnp.int32),
        pltpu.SMEM((2,), jnp.int32),
        pltpu.SemaphoreType.DMA,
        pltpu.SemaphoreType.DMA,
        pltpu.SemaphoreType.DMA,
        pltpu.SemaphoreType.DMA,
        pltpu.SemaphoreType.DMA,
    ),
)
def _sobel_topk(*args):
    _sobel_topk_body(*args)


def kernel(loudness, pitch):
    topk_idx, saliency = _sobel_topk(loudness.reshape(-1), pitch.reshape(-1))
    mask = jnp.zeros((B, K), dtype=jnp.bool_)
    return topk_idx.reshape(B, K), saliency.reshape(B, T), mask
